# baseline (speedup 1.0000x reference)
import functools

import jax
import jax.numpy as jnp
from jax import lax
from jax.experimental import pallas as pl
from jax.experimental.pallas import tpu as pltpu
from jax.experimental.pallas import tpu_sc as plsc

DIM = 32
N_EMBED = 8192
ROWS = 8192
BLK = 128
GRID = ROWS // BLK

_info = plsc.get_sparse_core_info()
NC, NS, L = _info.num_cores, _info.num_subcores, _info.num_lanes
NW = NC * NS
BPW = ROWS // NW
NCHUNK = 2
CHUNK = BPW // NCHUNK


def _argmax_body(embed_ref, ind_ref, idx_ref, table_ref):
    x = ind_ref[...]
    rowmax = jnp.max(x, axis=1, keepdims=True)
    iota = lax.broadcasted_iota(jnp.int32, x.shape, 1)
    idx_ref[0, 0, 0, :] = jnp.min(jnp.where(x == rowmax, iota, N_EMBED), axis=1)

    @pl.when(pl.program_id(0) == 0)
    def _():
        table_ref[...] = embed_ref[...].T


@jax.jit
def _run_argmax(ind, embed):
    return pl.pallas_call(
        _argmax_body,
        grid=(GRID,),
        in_specs=[
            pl.BlockSpec((DIM, N_EMBED), lambda i: (0, 0)),
            pl.BlockSpec((BLK, N_EMBED), lambda i: (i, 0)),
        ],
        out_specs=[
            pl.BlockSpec((1, 1, 1, BLK),
                         lambda i: (i // NCHUNK, i % NCHUNK, 0, 0)),
            pl.BlockSpec((N_EMBED, DIM), lambda i: (0, 0)),
        ],
        out_shape=[
            jax.ShapeDtypeStruct((NW, NCHUNK, 1, BLK), jnp.int32),
            jax.ShapeDtypeStruct((N_EMBED, DIM), jnp.float32),
        ],
    )(embed, ind)


_mesh = plsc.VectorSubcoreMesh(core_axis_name="c", subcore_axis_name="s")
_params = pltpu.CompilerParams(use_tc_tiling_on_sc=False,
                               needs_layout_passes=False)


@functools.partial(
    pl.kernel,
    mesh=_mesh,
    compiler_params=_params,
    out_type=[
        jax.ShapeDtypeStruct((ROWS, DIM), jnp.float32),
        jax.ShapeDtypeStruct((NW, L), jnp.float32),
    ],
    scratch_types=[
        pltpu.VMEM((NCHUNK, CHUNK), jnp.int32),
        pltpu.VMEM((BPW, DIM), jnp.float32),
        pltpu.VMEM((BPW, DIM), jnp.float32),
        pltpu.VMEM((L,), jnp.float32),
        pltpu.SemaphoreType.DMA,
    ],
)
def _sc_gather(table_hbm, idx_hbm, flat_hbm, q_hbm, part_hbm,
               idx_v, rows_v, flat_v, acc_v, sem):
    wid = lax.axis_index("s") * NC + lax.axis_index("c")
    base = wid * BPW
    pltpu.sync_copy(idx_hbm.at[wid], idx_v)
    pltpu.sync_copy(flat_hbm.at[pl.ds(base, BPW)], flat_v)
    copies = [
        pltpu.async_copy(table_hbm.at[idx_v.at[j]],
                         rows_v.at[pl.ds(j * CHUNK, CHUNK)], sem)
        for j in range(NCHUNK)
    ]
    for c in copies:
        c.wait()

    def body(i, accs):
        a0, a1 = accs
        for u in range(2):
            r0 = rows_v[2 * i + u, pl.ds(0, L)] - flat_v[2 * i + u, pl.ds(0, L)]
            r1 = rows_v[2 * i + u, pl.ds(L, L)] - flat_v[2 * i + u, pl.ds(L, L)]
            a0 = a0 + r0 * r0
            a1 = a1 + r1 * r1
        return a0, a1

    a0, a1 = lax.fori_loop(0, BPW // 2, body,
                           (jnp.zeros((L,), jnp.float32),
                            jnp.zeros((L,), jnp.float32)))
    acc_v[...] = a0 + a1
    pltpu.sync_copy(rows_v, q_hbm.at[pl.ds(base, BPW)])
    pltpu.sync_copy(acc_v, part_hbm.at[wid])


def kernel(input, ind, embed, fix):
    flatten = input.reshape(-1, DIM)
    idx4, table = _run_argmax(ind, embed)
    idx3 = idx4.reshape(NW, NCHUNK, CHUNK)
    q, part = _sc_gather(table, idx3, flatten)
    quantize = q.reshape(input.shape)
    embed_ind = idx3.reshape(input.shape[:-1])
    diff = (jnp.sum(part) / (ROWS * DIM)).astype(jnp.float32)
    return (quantize, diff, embed_ind)


# speedup vs baseline: 1.1728x; 1.1728x over previous
import functools

import jax
import jax.numpy as jnp
from jax import lax
from jax.experimental import pallas as pl
from jax.experimental.pallas import tpu as pltpu
from jax.experimental.pallas import tpu_sc as plsc

DIM = 32
N_EMBED = 8192
ROWS = 8192
BLK = 256
GRID = ROWS // BLK

_info = plsc.get_sparse_core_info()
NC, NS, L = _info.num_cores, _info.num_subcores, _info.num_lanes
NW = NC * NS
BPW = ROWS // NW
NCHUNK = 2
CHUNK = BPW // NCHUNK


def _argmax_body(ind_ref, idx_ref):
    x = ind_ref[...]
    rowmax = jnp.max(x, axis=1, keepdims=True)
    iota = lax.broadcasted_iota(jnp.int32, x.shape, 1)
    idx_ref[0, 0, :] = jnp.min(jnp.where(x == rowmax, iota, N_EMBED), axis=1)


@jax.jit
def _run_argmax(ind):
    return pl.pallas_call(
        _argmax_body,
        grid=(GRID,),
        in_specs=[pl.BlockSpec((BLK, N_EMBED), lambda i: (i, 0))],
        out_specs=pl.BlockSpec((1, 1, BLK), lambda i: (i, 0, 0)),
        out_shape=jax.ShapeDtypeStruct((GRID, 1, BLK), jnp.int32),
    )(ind)


_mesh = plsc.VectorSubcoreMesh(core_axis_name="c", subcore_axis_name="s")
_params = pltpu.CompilerParams(use_tc_tiling_on_sc=False,
                               needs_layout_passes=False)


@functools.partial(
    pl.kernel,
    mesh=_mesh,
    compiler_params=_params,
    out_type=[
        jax.ShapeDtypeStruct((ROWS, DIM), jnp.float32),
        jax.ShapeDtypeStruct((NW, L), jnp.float32),
    ],
    scratch_types=[
        pltpu.VMEM((NCHUNK, CHUNK), jnp.int32),
        pltpu.VMEM((BPW, DIM), jnp.float32),
        pltpu.VMEM((BPW, DIM), jnp.float32),
        pltpu.VMEM((L,), jnp.float32),
        pltpu.SemaphoreType.DMA,
    ],
)
def _sc_gather(table_hbm, idx_hbm, flat_hbm, q_hbm, part_hbm,
               idx_v, rows_v, flat_v, acc_v, sem):
    wid = lax.axis_index("s") * NC + lax.axis_index("c")
    base = wid * BPW
    pltpu.sync_copy(idx_hbm.at[wid], idx_v)
    pltpu.sync_copy(flat_hbm.at[pl.ds(base, BPW)], flat_v)
    copies = [
        pltpu.async_copy(table_hbm.at[idx_v.at[j]],
                         rows_v.at[pl.ds(j * CHUNK, CHUNK)], sem)
        for j in range(NCHUNK)
    ]
    for c in copies:
        c.wait()

    def body(i, accs):
        a0, a1 = accs
        for u in range(2):
            r0 = rows_v[2 * i + u, pl.ds(0, L)] - flat_v[2 * i + u, pl.ds(0, L)]
            r1 = rows_v[2 * i + u, pl.ds(L, L)] - flat_v[2 * i + u, pl.ds(L, L)]
            a0 = a0 + r0 * r0
            a1 = a1 + r1 * r1
        return a0, a1

    a0, a1 = lax.fori_loop(0, BPW // 2, body,
                           (jnp.zeros((L,), jnp.float32),
                            jnp.zeros((L,), jnp.float32)))
    acc_v[...] = a0 + a1
    pltpu.sync_copy(rows_v, q_hbm.at[pl.ds(base, BPW)])
    pltpu.sync_copy(acc_v, part_hbm.at[wid])


def kernel(input, ind, embed, fix):
    flatten = input.reshape(-1, DIM)
    idx4 = _run_argmax(ind)
    table = embed.T
    idx3 = idx4.reshape(NW, NCHUNK, CHUNK)
    q, part = _sc_gather(table, idx3, flatten)
    quantize = q.reshape(input.shape)
    embed_ind = idx3.reshape(input.shape[:-1])
    diff = (jnp.sum(part) / (ROWS * DIM)).astype(jnp.float32)
    return (quantize, diff, embed_ind)


# speedup vs baseline: 1.2252x; 1.0447x over previous
import functools

import jax
import jax.numpy as jnp
from jax import lax
from jax.experimental import pallas as pl
from jax.experimental.pallas import tpu as pltpu
from jax.experimental.pallas import tpu_sc as plsc

DIM = 32
N_EMBED = 8192
ROWS = 8192
BLK = 512
GRID = ROWS // BLK

_info = plsc.get_sparse_core_info()
NC, NS, L = _info.num_cores, _info.num_subcores, _info.num_lanes
NW = NC * NS
BPW = ROWS // NW
NCHUNK = 2
CHUNK = BPW // NCHUNK


def _argmax_body(ind_ref, idx_ref):
    x = ind_ref[...]
    rowmax = jnp.max(x, axis=1, keepdims=True)
    iota = lax.broadcasted_iota(jnp.int32, x.shape, 1)
    idx_ref[0, 0, :] = jnp.min(jnp.where(x == rowmax, iota, N_EMBED), axis=1)


@jax.jit
def _run_argmax(ind):
    return pl.pallas_call(
        _argmax_body,
        grid=(GRID,),
        in_specs=[pl.BlockSpec((BLK, N_EMBED), lambda i: (i, 0))],
        out_specs=pl.BlockSpec((1, 1, BLK), lambda i: (i, 0, 0)),
        out_shape=jax.ShapeDtypeStruct((GRID, 1, BLK), jnp.int32),
    )(ind)


_mesh = plsc.VectorSubcoreMesh(core_axis_name="c", subcore_axis_name="s")
_params = pltpu.CompilerParams(use_tc_tiling_on_sc=False,
                               needs_layout_passes=False)


@functools.partial(
    pl.kernel,
    mesh=_mesh,
    compiler_params=_params,
    out_type=[
        jax.ShapeDtypeStruct((ROWS, DIM), jnp.float32),
        jax.ShapeDtypeStruct((NW, L), jnp.float32),
    ],
    scratch_types=[
        pltpu.VMEM((NCHUNK, CHUNK), jnp.int32),
        pltpu.VMEM((BPW, DIM), jnp.float32),
        pltpu.VMEM((BPW, DIM), jnp.float32),
        pltpu.VMEM((L,), jnp.float32),
        pltpu.SemaphoreType.DMA,
    ],
)
def _sc_gather(table_hbm, idx_hbm, flat_hbm, q_hbm, part_hbm,
               idx_v, rows_v, flat_v, acc_v, sem):
    wid = lax.axis_index("s") * NC + lax.axis_index("c")
    base = wid * BPW
    pltpu.sync_copy(idx_hbm.at[wid], idx_v)
    pltpu.sync_copy(flat_hbm.at[pl.ds(base, BPW)], flat_v)
    copies = [
        pltpu.async_copy(table_hbm.at[idx_v.at[j]],
                         rows_v.at[pl.ds(j * CHUNK, CHUNK)], sem)
        for j in range(NCHUNK)
    ]
    for c in copies:
        c.wait()

    def body(i, accs):
        a0, a1 = accs
        for u in range(2):
            r0 = rows_v[2 * i + u, pl.ds(0, L)] - flat_v[2 * i + u, pl.ds(0, L)]
            r1 = rows_v[2 * i + u, pl.ds(L, L)] - flat_v[2 * i + u, pl.ds(L, L)]
            a0 = a0 + r0 * r0
            a1 = a1 + r1 * r1
        return a0, a1

    a0, a1 = lax.fori_loop(0, BPW // 2, body,
                           (jnp.zeros((L,), jnp.float32),
                            jnp.zeros((L,), jnp.float32)))
    acc_v[...] = a0 + a1
    pltpu.sync_copy(rows_v, q_hbm.at[pl.ds(base, BPW)])
    pltpu.sync_copy(acc_v, part_hbm.at[wid])


def kernel(input, ind, embed, fix):
    flatten = input.reshape(-1, DIM)
    idx4 = _run_argmax(ind)
    table = embed.T
    idx3 = idx4.reshape(NW, NCHUNK, CHUNK)
    q, part = _sc_gather(table, idx3, flatten)
    quantize = q.reshape(input.shape)
    embed_ind = idx3.reshape(input.shape[:-1])
    diff = (jnp.sum(part) / (ROWS * DIM)).astype(jnp.float32)
    return (quantize, diff, embed_ind)
